# Initial kernel scaffold; baseline (speedup 1.0000x reference)
import functools

import jax
import jax.numpy as jnp
from jax import lax
from jax.experimental import pallas as pl
from jax.experimental.pallas import tpu as pltpu
from jax.experimental.pallas import tpu_sc as plsc

_N_NODES = 10000
_N_EDGES = 320000
_D = 128
_NSEG = 10000
_NC = 2
_NS = 16
_NW = _NC * _NS
_K = 128
_G = _N_EDGES // _K
_NITER = -(-_G // _NW)
_HALF = (_NITER + 1) // 2
_RPT = _NSEG // _NS


def _sc_body(x_hbm, tn_hbm, ti_hbm, z0_hbm, out_hbm,
             idxn0, idxn1, idxs0, idxs1, rows0, rows1, zsh,
             sin0, sin1, sis0, sis1, sg0, sg1, ss0, ss1):
    c = lax.axis_index("c")
    s = lax.axis_index("s")
    wid = s * _NC + c
    idxn = (idxn0, idxn1)
    idxs = (idxs0, idxs1)
    rows = (rows0, rows1)
    sin = (sin0, sin1)
    sis = (sis0, sis1)
    sg = (sg0, sg1)
    ss = (ss0, ss1)

    r0 = s * _RPT
    pltpu.sync_copy(z0_hbm.at[pl.ds(r0, _RPT)], zsh.at[pl.ds(r0, _RPT)])
    plsc.subcore_barrier()

    def step(t, carry):
        for b in range(2):
            cc = 2 * t + b
            g = wid + _NW * cc

            @pl.when(g < _G)
            def _chunk():
                @pl.when(cc >= 2)
                def _drain_prev():
                    pltpu.make_async_copy(rows[b], zsh.at[idxs[b]], ss[b]).wait()

                e0 = g * _K
                cin = pltpu.async_copy(tn_hbm.at[pl.ds(e0, _K)], idxn[b], sin[b])
                cis = pltpu.async_copy(ti_hbm.at[pl.ds(e0, _K)], idxs[b], sis[b])
                cin.wait()
                cis.wait()
                pltpu.async_copy(x_hbm.at[idxn[b]], rows[b], sg[b]).wait()
                pltpu.async_copy(rows[b], zsh.at[idxs[b]], ss[b], add=True)

        return carry

    lax.fori_loop(0, _HALF, step, 0)

    for b in range(2):
        pltpu.make_async_copy(rows[b], zsh.at[idxs[b]], ss[b]).wait()
    plsc.subcore_barrier()

    pltpu.sync_copy(zsh.at[pl.ds(r0, _RPT)],
                    out_hbm.at[pl.ds(c * _NSEG + r0, _RPT)])


_sc_segment_sum = functools.partial(
    pl.kernel,
    out_type=jax.ShapeDtypeStruct((_NC * _NSEG, _D), jnp.float32),
    mesh=plsc.VectorSubcoreMesh(core_axis_name="c", subcore_axis_name="s",
                                num_cores=_NC, num_subcores=_NS),
    scratch_types=[
        pltpu.VMEM((_K,), jnp.int32),
        pltpu.VMEM((_K,), jnp.int32),
        pltpu.VMEM((_K,), jnp.int32),
        pltpu.VMEM((_K,), jnp.int32),
        pltpu.VMEM((_K, _D), jnp.float32),
        pltpu.VMEM((_K, _D), jnp.float32),
        pltpu.VMEM_SHARED((_NSEG, _D), jnp.float32),
        pltpu.SemaphoreType.DMA,
        pltpu.SemaphoreType.DMA,
        pltpu.SemaphoreType.DMA,
        pltpu.SemaphoreType.DMA,
        pltpu.SemaphoreType.DMA,
        pltpu.SemaphoreType.DMA,
        pltpu.SemaphoreType.DMA,
        pltpu.SemaphoreType.DMA,
    ],
)(_sc_body)


def _mlp_body(zp_ref, w1_ref, b1_ref, w2t_ref, b2_ref, o_ref):
    z = zp_ref[0:_NSEG, :] + zp_ref[_NSEG:2 * _NSEG, :]
    h = jnp.dot(z, w1_ref[...], preferred_element_type=jnp.float32)
    h = jnp.maximum(h + b1_ref[...], 0.0)
    logit = jnp.sum(h * w2t_ref[...], axis=1, keepdims=True) + b2_ref[...]
    o_ref[...] = jax.nn.sigmoid(logit)


_mlp = pl.pallas_call(
    _mlp_body,
    out_shape=jax.ShapeDtypeStruct((_NSEG, 1), jnp.float32),
)


def kernel(x, target_nodes, target_ids, W1, b1, W2, b2):
    tn = target_nodes.astype(jnp.int32)
    ti = target_ids.astype(jnp.int32)
    zeros = jnp.zeros((_NSEG, _D), jnp.float32)
    zparts = _sc_segment_sum(x, tn, ti, zeros)
    out = _mlp(zparts, W1, b1.reshape(1, _D), W2.reshape(1, _D),
               b2.reshape(1, 1))
    return out.reshape(_NSEG)


# speedup vs baseline: 9.5672x; 9.5672x over previous
import functools

import jax
import jax.numpy as jnp
from jax import lax
from jax.experimental import pallas as pl
from jax.experimental.pallas import tpu as pltpu
from jax.experimental.pallas import tpu_sc as plsc

_N_NODES = 10000
_N_EDGES = 320000
_D = 128
_NSEG = 10000
_NC = 2
_NS = 16
_NW = _NC * _NS
_K = 128
_G = _N_EDGES // _K
_NITER = -(-_G // _NW)
_HALF = (_NITER + 1) // 2
_RPT = 624


def _sc_body(x_hbm, tn_hbm, ti_hbm, z0_hbm, out_hbm,
             idxn0, idxn1, idxs0, idxs1, rows0, rows1, zsh,
             sin0, sin1, sis0, sis1, sg0, sg1, ss0, ss1):
    c = lax.axis_index("c")
    s = lax.axis_index("s")
    wid = s * _NC + c
    idxn = (idxn0, idxn1)
    idxs = (idxs0, idxs1)
    rows = (rows0, rows1)
    sin = (sin0, sin1)
    sis = (sis0, sis1)
    sg = (sg0, sg1)
    ss = (ss0, ss1)

    r0 = s * _RPT
    _RPT_LAST = _NSEG - (_NS - 1) * _RPT

    @pl.when(s < _NS - 1)
    def _zero_main():
        pltpu.sync_copy(z0_hbm.at[pl.ds(r0, _RPT)], zsh.at[pl.ds(r0, _RPT)])

    @pl.when(s == _NS - 1)
    def _zero_last():
        pltpu.sync_copy(z0_hbm.at[pl.ds(r0, _RPT_LAST)],
                        zsh.at[pl.ds(r0, _RPT_LAST)])

    plsc.subcore_barrier()

    def step(t, carry):
        for b in range(2):
            cc = 2 * t + b
            g = wid + _NW * cc

            @pl.when(g < _G)
            def _chunk():
                @pl.when(cc >= 2)
                def _drain_prev():
                    pltpu.make_async_copy(rows[b], zsh.at[idxs[b]], ss[b]).wait()

                e0 = g * _K
                cin = pltpu.async_copy(tn_hbm.at[pl.ds(e0, _K)], idxn[b], sin[b])
                cis = pltpu.async_copy(ti_hbm.at[pl.ds(e0, _K)], idxs[b], sis[b])
                cin.wait()
                cis.wait()
                pltpu.async_copy(x_hbm.at[idxn[b]], rows[b], sg[b]).wait()
                pltpu.async_copy(rows[b], zsh.at[idxs[b]], ss[b], add=True)

        return carry

    lax.fori_loop(0, _HALF, step, 0)

    for b in range(2):
        pltpu.make_async_copy(rows[b], zsh.at[idxs[b]], ss[b]).wait()
    plsc.subcore_barrier()

    @pl.when(s < _NS - 1)
    def _out_main():
        pltpu.sync_copy(zsh.at[pl.ds(r0, _RPT)],
                        out_hbm.at[pl.ds(c * _NSEG + r0, _RPT)])

    @pl.when(s == _NS - 1)
    def _out_last():
        pltpu.sync_copy(zsh.at[pl.ds(r0, _RPT_LAST)],
                        out_hbm.at[pl.ds(c * _NSEG + r0, _RPT_LAST)])


_sc_segment_sum = functools.partial(
    pl.kernel,
    out_type=jax.ShapeDtypeStruct((_NC * _NSEG, _D), jnp.float32),
    mesh=plsc.VectorSubcoreMesh(core_axis_name="c", subcore_axis_name="s",
                                num_cores=_NC, num_subcores=_NS),
    scratch_types=[
        pltpu.VMEM((_K,), jnp.int32),
        pltpu.VMEM((_K,), jnp.int32),
        pltpu.VMEM((_K,), jnp.int32),
        pltpu.VMEM((_K,), jnp.int32),
        pltpu.VMEM((_K, _D), jnp.float32),
        pltpu.VMEM((_K, _D), jnp.float32),
        pltpu.VMEM_SHARED((_NSEG, _D), jnp.float32),
        pltpu.SemaphoreType.DMA,
        pltpu.SemaphoreType.DMA,
        pltpu.SemaphoreType.DMA,
        pltpu.SemaphoreType.DMA,
        pltpu.SemaphoreType.DMA,
        pltpu.SemaphoreType.DMA,
        pltpu.SemaphoreType.DMA,
        pltpu.SemaphoreType.DMA,
    ],
)(_sc_body)


def _mlp_body(zp_ref, w1_ref, b1_ref, w2t_ref, b2_ref, o_ref):
    z = zp_ref[0:_NSEG, :] + zp_ref[_NSEG:2 * _NSEG, :]
    h = jnp.dot(z, w1_ref[...], preferred_element_type=jnp.float32)
    h = jnp.maximum(h + b1_ref[...], 0.0)
    logit = jnp.sum(h * w2t_ref[...], axis=1, keepdims=True) + b2_ref[...]
    o_ref[...] = jax.nn.sigmoid(logit)


_mlp = pl.pallas_call(
    _mlp_body,
    out_shape=jax.ShapeDtypeStruct((_NSEG, 1), jnp.float32),
)


def kernel(x, target_nodes, target_ids, W1, b1, W2, b2):
    tn = target_nodes.astype(jnp.int32)
    ti = target_ids.astype(jnp.int32)
    zeros = jnp.zeros((_NSEG, _D), jnp.float32)
    zparts = _sc_segment_sum(x, tn, ti, zeros)
    out = _mlp(zparts, W1, b1.reshape(1, _D), W2.reshape(1, _D),
               b2.reshape(1, 1))
    return out.reshape(_NSEG)


# speedup vs baseline: 11.6530x; 1.2180x over previous
import functools

import jax
import jax.numpy as jnp
from jax import lax
from jax.experimental import pallas as pl
from jax.experimental.pallas import tpu as pltpu
from jax.experimental.pallas import tpu_sc as plsc

_N_NODES = 10000
_N_EDGES = 320000
_D = 128
_NSEG = 10000
_NC = 2
_NS = 16
_NW = _NC * _NS
_K = 128
_G = _N_EDGES // _K
_NITER = -(-_G // _NW)
_HALF = (_NITER + 1) // 2
_RPT = 624


def _sc_body(x_hbm, tn_hbm, ti_hbm, z0_hbm, out_hbm,
             idxn0, idxn1, idxs0, idxs1, rows0, rows1, zsh,
             sin0, sin1, sis0, sis1, sg0, sg1, ss0, ss1):
    c = lax.axis_index("c")
    s = lax.axis_index("s")
    wid = s * _NC + c
    idxn = (idxn0, idxn1)
    idxs = (idxs0, idxs1)
    rows = (rows0, rows1)
    sin = (sin0, sin1)
    sis = (sis0, sis1)
    sg = (sg0, sg1)
    ss = (ss0, ss1)

    r0 = s * _RPT
    _RPT_LAST = _NSEG - (_NS - 1) * _RPT

    @pl.when(s < _NS - 1)
    def _zero_main():
        pltpu.sync_copy(z0_hbm.at[pl.ds(0, _RPT)], zsh.at[pl.ds(r0, _RPT)])

    @pl.when(s == _NS - 1)
    def _zero_last():
        pltpu.sync_copy(z0_hbm.at[pl.ds(0, _RPT_LAST)],
                        zsh.at[pl.ds(r0, _RPT_LAST)])

    plsc.subcore_barrier()

    def step(t, carry):
        for b in range(2):
            cc = 2 * t + b
            g = wid + _NW * cc

            @pl.when(g < _G)
            def _chunk():
                e0 = g * _K

                @pl.when(cc >= 2)
                def _drain_prev():
                    pltpu.make_async_copy(rows[b], zsh.at[idxs[b]], ss[b]).wait()

                cis = pltpu.async_copy(ti_hbm.at[pl.ds(e0, _K)], idxs[b], sis[b])

                @pl.when(cc < 2)
                def _first_idxn():
                    pltpu.async_copy(tn_hbm.at[pl.ds(e0, _K)], idxn[b], sin[b])

                pltpu.make_async_copy(tn_hbm.at[pl.ds(e0, _K)], idxn[b],
                                      sin[b]).wait()
                pltpu.async_copy(x_hbm.at[idxn[b]], rows[b], sg[b]).wait()
                cis.wait()
                pltpu.async_copy(rows[b], zsh.at[idxs[b]], ss[b], add=True)

                g2 = g + 2 * _NW

                @pl.when(g2 < _G)
                def _prefetch_idxn():
                    pltpu.async_copy(tn_hbm.at[pl.ds(g2 * _K, _K)], idxn[b],
                                     sin[b])

        return carry

    lax.fori_loop(0, _HALF, step, 0)

    for b in range(2):
        pltpu.make_async_copy(rows[b], zsh.at[idxs[b]], ss[b]).wait()
    plsc.subcore_barrier()

    @pl.when(s < _NS - 1)
    def _out_main():
        pltpu.sync_copy(zsh.at[pl.ds(r0, _RPT)],
                        out_hbm.at[pl.ds(c * _NSEG + r0, _RPT)])

    @pl.when(s == _NS - 1)
    def _out_last():
        pltpu.sync_copy(zsh.at[pl.ds(r0, _RPT_LAST)],
                        out_hbm.at[pl.ds(c * _NSEG + r0, _RPT_LAST)])


_sc_segment_sum = functools.partial(
    pl.kernel,
    out_type=jax.ShapeDtypeStruct((_NC * _NSEG, _D), jnp.float32),
    mesh=plsc.VectorSubcoreMesh(core_axis_name="c", subcore_axis_name="s",
                                num_cores=_NC, num_subcores=_NS),
    scratch_types=[
        pltpu.VMEM((_K,), jnp.int32),
        pltpu.VMEM((_K,), jnp.int32),
        pltpu.VMEM((_K,), jnp.int32),
        pltpu.VMEM((_K,), jnp.int32),
        pltpu.VMEM((_K, _D), jnp.float32),
        pltpu.VMEM((_K, _D), jnp.float32),
        pltpu.VMEM_SHARED((_NSEG, _D), jnp.float32),
        pltpu.SemaphoreType.DMA,
        pltpu.SemaphoreType.DMA,
        pltpu.SemaphoreType.DMA,
        pltpu.SemaphoreType.DMA,
        pltpu.SemaphoreType.DMA,
        pltpu.SemaphoreType.DMA,
        pltpu.SemaphoreType.DMA,
        pltpu.SemaphoreType.DMA,
    ],
)(_sc_body)


def _mlp_body(zp_ref, w1_ref, b1_ref, w2t_ref, b2_ref, o_ref):
    z = zp_ref[0:_NSEG, :] + zp_ref[_NSEG:2 * _NSEG, :]
    h = jnp.dot(z, w1_ref[...], preferred_element_type=jnp.float32)
    h = jnp.maximum(h + b1_ref[...], 0.0)
    logit = jnp.sum(h * w2t_ref[...], axis=1, keepdims=True) + b2_ref[...]
    o_ref[...] = jax.nn.sigmoid(logit)


_mlp = pl.pallas_call(
    _mlp_body,
    out_shape=jax.ShapeDtypeStruct((_NSEG, 1), jnp.float32),
)


def kernel(x, target_nodes, target_ids, W1, b1, W2, b2):
    tn = target_nodes.astype(jnp.int32)
    ti = target_ids.astype(jnp.int32)
    zeros = jnp.zeros((_NSEG - (_NS - 1) * _RPT, _D), jnp.float32)
    zparts = _sc_segment_sum(x, tn, ti, zeros)
    out = _mlp(zparts, W1, b1.reshape(1, _D), W2.reshape(1, _D),
               b2.reshape(1, 1))
    return out.reshape(_NSEG)
